# baseline (speedup 1.0000x reference)
import functools

import jax
import jax.numpy as jnp
from jax import lax
from jax.experimental import pallas as pl
from jax.experimental.pallas import tpu as pltpu
from jax.experimental.pallas import tpu_sc as plsc

_NUM_CORES = 2
_NUM_SUBCORES = 16
_NW = _NUM_CORES * _NUM_SUBCORES


@functools.lru_cache(maxsize=None)
def _make_gather(batch: int, hist: int, D: int, nbuf: int):
    assert batch % _NW == 0
    rows_per_w = batch // _NW
    assert rows_per_w % nbuf == 0
    mesh = plsc.VectorSubcoreMesh(core_axis_name="c", subcore_axis_name="s")

    @functools.partial(
        pl.kernel,
        out_type=jax.ShapeDtypeStruct((batch, hist, D), jnp.float32),
        mesh=mesh,
        scratch_types=[
            pltpu.VMEM((rows_per_w, hist), jnp.int32),
            pltpu.VMEM((nbuf, hist, D), jnp.float32),
            pltpu.SemaphoreType.DMA,
            pltpu.SemaphoreType.DMA,
        ],
        compiler_params=pltpu.CompilerParams(use_tc_tiling_on_sc=False),
    )
    def gather_kernel(idx_hbm, table_hbm, out_hbm, idx_v, rows_v, sem_g, sem_o):
        wid = lax.axis_index("s") * _NUM_CORES + lax.axis_index("c")
        base = wid * rows_per_w
        pltpu.sync_copy(idx_hbm.at[pl.ds(base, rows_per_w), :], idx_v)

        def gather_start(j, b):
            pltpu.async_copy(table_hbm.at[idx_v.at[j]], rows_v.at[b], sem_g)

        def gather_wait(b):
            pltpu.make_async_copy(table_hbm.at[idx_v.at[0]], rows_v.at[b], sem_g).wait()

        def out_start(j, b):
            pltpu.async_copy(rows_v.at[b], out_hbm.at[base + j], sem_o)

        def out_wait(b):
            pltpu.make_async_copy(rows_v.at[b], out_hbm.at[base], sem_o).wait()

        for r in range(nbuf):
            gather_start(r, r)

        @pl.loop(0, rows_per_w, step=nbuf)
        def _outer(g):
            for b in range(nbuf):
                j = g + b
                gather_wait(b)
                out_start(j, b)

                @pl.when(j >= 1)
                def _():
                    out_wait((b - 1) % nbuf)

                    @pl.when(j - 1 + nbuf < rows_per_w)
                    def _():
                        gather_start(j - 1 + nbuf, (b - 1) % nbuf)

        out_wait((rows_per_w - 1) % nbuf)

    return gather_kernel


def kernel(indices, table):
    batch, hist = indices.shape
    num_codes, dim = table.shape
    return _make_gather(batch, hist, dim, 8)(indices, table)
